# baseline (speedup 1.0000x reference)
import functools

import jax
import jax.numpy as jnp
from jax import lax
from jax.experimental import pallas as pl
from jax.experimental.pallas import tpu as pltpu
from jax.experimental.pallas import tpu_sc as plsc

_N_FEAT = 131072
_N_PROBES = 8192
_N_CAND = 512
_D_MODEL = 4096

_N_SPLIT = 2
_P_SPLIT = _N_PROBES // _N_SPLIT

_NC = 2
_NS = 16
_L = 16
_NW = _NC * _NS
_P_PER_W = _P_SPLIT // _NW
_CHUNK = 16
_N_CHUNKS = _P_PER_W // _CHUNK
_G = _N_CAND // _L


def _probe_values_sc(packed_table, probe_candidates, selection_logits, split):
  mesh = plsc.VectorSubcoreMesh(core_axis_name="c", subcore_axis_name="s")

  @functools.partial(
      pl.kernel,
      mesh=mesh,
      out_type=jax.ShapeDtypeStruct((_P_SPLIT,), jnp.float32),
      compiler_params=pltpu.CompilerParams(needs_layout_passes=False),
      cost_estimate=pl.CostEstimate(
          flops=8 * _P_SPLIT * _N_CAND,
          bytes_accessed=2 * 4 * _P_SPLIT * _N_CAND + _N_FEAT * 2,
          transcendentals=_P_SPLIT * _N_CAND,
      ),
      scratch_types=[
          pltpu.VMEM((_N_FEAT // 2,), jnp.int32),
          pltpu.VMEM((2, _CHUNK, _N_CAND), jnp.int32),
          pltpu.VMEM((2, _CHUNK, _N_CAND), jnp.float32),
          pltpu.VMEM((_P_PER_W,), jnp.float32),
          pltpu.SemaphoreType.DMA,
          pltpu.SemaphoreType.DMA,
      ],
  )
  def body(table_hbm, idx_hbm, logit_hbm, out_hbm,
           table_v, idx_v, log_v, pv_v, sem0, sem1):
    wid = lax.axis_index("s") * _NC + lax.axis_index("c")
    base = wid * _P_PER_W
    src_base = split * _P_SPLIT + base
    sems = (sem0, sem1)
    lane = lax.broadcasted_iota(jnp.int32, (_L,), 0)
    pltpu.sync_copy(table_hbm, table_v)

    def start_fetch(c, b):
      row0 = src_base + c * _CHUNK
      pltpu.async_copy(idx_hbm.at[pl.ds(row0, _CHUNK), :], idx_v.at[b], sems[b])
      pltpu.async_copy(logit_hbm.at[pl.ds(row0, _CHUNK), :], log_v.at[b],
                       sems[b])

    def wait_fetch(c, b):
      row0 = src_base + c * _CHUNK
      pltpu.make_async_copy(idx_hbm.at[pl.ds(row0, _CHUNK), :], idx_v.at[b],
                            sems[b]).wait()
      pltpu.make_async_copy(logit_hbm.at[pl.ds(row0, _CHUNK), :], log_v.at[b],
                            sems[b]).wait()

    def compute_chunk(c, b):
      def probe_body(p, carry):
        acc = jnp.zeros((_L,), jnp.float32)
        wsum = jnp.zeros((_L,), jnp.float32)
        for g in range(_G):
          lg = log_v[b, p, pl.ds(g * _L, _L)]
          e = jnp.exp(lg)
          iv = idx_v[b, p, pl.ds(g * _L, _L)]
          widx = lax.shift_right_logical(iv, 1)
          wbits = plsc.load_gather(table_v, [widx])
          odd = lax.bitwise_and(iv, 1) == 1
          bits = jnp.where(odd, wbits, lax.shift_left(wbits, 16))
          bits = lax.bitwise_and(bits, jnp.int32(-65536))
          val = lax.bitcast_convert_type(bits, jnp.float32)
          acc = acc + e * val
          wsum = wsum + e
        num = jnp.broadcast_to(jnp.sum(acc), (_L,))
        den = jnp.broadcast_to(jnp.sum(wsum), (_L,))
        plsc.store_scatter(
            pv_v,
            [jnp.broadcast_to(c * _CHUNK + p, (_L,)).astype(jnp.int32)],
            num / den,
            mask=lane == 0,
        )
        return carry

      lax.fori_loop(0, _CHUNK, probe_body, 0)

    start_fetch(0, 0)

    def pair_body(h, carry):
      c0 = 2 * h
      c1 = c0 + 1
      wait_fetch(c0, 0)
      start_fetch(c1, 1)
      compute_chunk(c0, 0)
      wait_fetch(c1, 1)

      @pl.when(h < _N_CHUNKS // 2 - 1)
      def _():
        start_fetch(c1 + 1, 0)

      compute_chunk(c1, 1)
      return carry

    lax.fori_loop(0, _N_CHUNKS // 2, pair_body, 0)

    pltpu.sync_copy(pv_v, out_hbm.at[pl.ds(base, _P_PER_W)])

  return body(packed_table, probe_candidates, selection_logits)


_PB = 1024
_NB = _P_SPLIT // _PB


def _steer_tc(acc, pv, state, W1, b1, W2, b2, dirs, split, last):
  def body(acc_in_ref, state_ref, w1_ref, b1_ref, pv_ref, w2_ref, b2_ref,
           dirs_ref, out_ref):
    i = pl.program_id(0)
    st = state_ref[...]
    z = jnp.sum(w1_ref[...] * st, axis=1) + b1_ref[0, :]
    h = 0.5 * z * (1.0 + lax.erf(z * jnp.float32(0.7071067811865476)))
    m = jnp.sum(w2_ref[...] * h[None, :], axis=1) + b2_ref[0, :]
    wvec = pv_ref[0, :] * jax.nn.sigmoid(m)
    contrib = jnp.dot(wvec[None, :], dirs_ref[...],
                      preferred_element_type=jnp.float32)

    @pl.when(i == 0)
    def _():
      out_ref[...] = acc_in_ref[...]

    out_ref[...] += contrib

    if last:
      @pl.when(i == _NB - 1)
      def _():
        out_ref[...] = jnp.tanh(out_ref[...])

  row0 = split * _NB

  return pl.pallas_call(
      body,
      grid=(_NB,),
      in_specs=[
          pl.BlockSpec((1, _D_MODEL), lambda i: (0, 0)),
          pl.BlockSpec((1, 4), lambda i: (0, 0)),
          pl.BlockSpec((32, 4), lambda i: (0, 0)),
          pl.BlockSpec((1, 32), lambda i: (0, 0)),
          pl.BlockSpec((1, _PB), lambda i: (0, i)),
          pl.BlockSpec((_PB, 32), lambda i: (row0 + i, 0)),
          pl.BlockSpec((1, _PB), lambda i: (0, row0 + i)),
          pl.BlockSpec((_PB, _D_MODEL), lambda i: (row0 + i, 0)),
      ],
      out_specs=pl.BlockSpec((1, _D_MODEL), lambda i: (0, 0)),
      out_shape=jax.ShapeDtypeStruct((1, _D_MODEL), jnp.float32),
      compiler_params=pltpu.CompilerParams(
          dimension_semantics=("arbitrary",)),
      cost_estimate=pl.CostEstimate(
          flops=2 * _P_SPLIT * _D_MODEL,
          bytes_accessed=4 * _P_SPLIT * _D_MODEL,
          transcendentals=0,
      ),
  )(acc, state.reshape(1, 4), W1, b1.reshape(1, 32), pv.reshape(1, _P_SPLIT),
    W2, b2.reshape(1, _N_PROBES), dirs)


def kernel(activations, state, probe_candidates, selection_logits,
           steering_dirs, W1, b1, W2, b2, scale):
  acts_bf = activations.astype(jnp.bfloat16)
  packed = lax.bitcast_convert_type(
      acts_bf.reshape(_N_FEAT // 2, 2), jnp.int32)
  acc = jnp.zeros((1, _D_MODEL), jnp.float32)
  for k in range(_N_SPLIT):
    pv = _probe_values_sc(packed, probe_candidates, selection_logits, k)
    acc = _steer_tc(acc, pv, state, W1, b1, W2, b2, steering_dirs,
                    split=k, last=(k == _N_SPLIT - 1))
  return acc.reshape(_D_MODEL) * (scale * 10.0)


# speedup vs baseline: 1.0183x; 1.0183x over previous
import functools

import jax
import jax.numpy as jnp
from jax import lax
from jax.experimental import pallas as pl
from jax.experimental.pallas import tpu as pltpu
from jax.experimental.pallas import tpu_sc as plsc

_N_FEAT = 131072
_N_PROBES = 8192
_N_CAND = 512
_D_MODEL = 4096

_N_SPLIT = 1
_P_SPLIT = _N_PROBES // _N_SPLIT

_NC = 2
_NS = 16
_L = 16
_NW = _NC * _NS
_P_PER_W = _P_SPLIT // _NW
_CHUNK = 16
_N_CHUNKS = _P_PER_W // _CHUNK
_G = _N_CAND // _L


def _probe_values_sc(packed_table, probe_candidates, selection_logits, split):
  mesh = plsc.VectorSubcoreMesh(core_axis_name="c", subcore_axis_name="s")

  @functools.partial(
      pl.kernel,
      mesh=mesh,
      out_type=jax.ShapeDtypeStruct((_P_SPLIT,), jnp.float32),
      compiler_params=pltpu.CompilerParams(needs_layout_passes=False),
      cost_estimate=pl.CostEstimate(
          flops=8 * _P_SPLIT * _N_CAND,
          bytes_accessed=2 * 4 * _P_SPLIT * _N_CAND + _N_FEAT * 2,
          transcendentals=_P_SPLIT * _N_CAND,
      ),
      scratch_types=[
          pltpu.VMEM((_N_FEAT // 2,), jnp.int32),
          pltpu.VMEM((2, _CHUNK, _N_CAND), jnp.int32),
          pltpu.VMEM((2, _CHUNK, _N_CAND), jnp.float32),
          pltpu.VMEM((_P_PER_W,), jnp.float32),
          pltpu.SemaphoreType.DMA,
          pltpu.SemaphoreType.DMA,
      ],
  )
  def body(table_hbm, idx_hbm, logit_hbm, out_hbm,
           table_v, idx_v, log_v, pv_v, sem0, sem1):
    wid = lax.axis_index("s") * _NC + lax.axis_index("c")
    base = wid * _P_PER_W
    src_base = split * _P_SPLIT + base
    sems = (sem0, sem1)
    lane = lax.broadcasted_iota(jnp.int32, (_L,), 0)
    pltpu.sync_copy(table_hbm, table_v)

    def start_fetch(c, b):
      row0 = src_base + c * _CHUNK
      pltpu.async_copy(idx_hbm.at[pl.ds(row0, _CHUNK), :], idx_v.at[b], sems[b])
      pltpu.async_copy(logit_hbm.at[pl.ds(row0, _CHUNK), :], log_v.at[b],
                       sems[b])

    def wait_fetch(c, b):
      row0 = src_base + c * _CHUNK
      pltpu.make_async_copy(idx_hbm.at[pl.ds(row0, _CHUNK), :], idx_v.at[b],
                            sems[b]).wait()
      pltpu.make_async_copy(logit_hbm.at[pl.ds(row0, _CHUNK), :], log_v.at[b],
                            sems[b]).wait()

    def compute_chunk(c, b):
      def probe_body(p, carry):
        acc = jnp.zeros((_L,), jnp.float32)
        wsum = jnp.zeros((_L,), jnp.float32)
        for g in range(_G):
          lg = log_v[b, p, pl.ds(g * _L, _L)]
          e = jnp.exp(lg)
          iv = idx_v[b, p, pl.ds(g * _L, _L)]
          widx = lax.shift_right_logical(iv, 1)
          wbits = plsc.load_gather(table_v, [widx])
          odd = lax.bitwise_and(iv, 1) == 1
          bits = jnp.where(odd, wbits, lax.shift_left(wbits, 16))
          bits = lax.bitwise_and(bits, jnp.int32(-65536))
          val = lax.bitcast_convert_type(bits, jnp.float32)
          acc = acc + e * val
          wsum = wsum + e
        num = jnp.broadcast_to(jnp.sum(acc), (_L,))
        den = jnp.broadcast_to(jnp.sum(wsum), (_L,))
        plsc.store_scatter(
            pv_v,
            [jnp.broadcast_to(c * _CHUNK + p, (_L,)).astype(jnp.int32)],
            num / den,
            mask=lane == 0,
        )
        return carry

      lax.fori_loop(0, _CHUNK, probe_body, 0)

    start_fetch(0, 0)

    def pair_body(h, carry):
      c0 = 2 * h
      c1 = c0 + 1
      wait_fetch(c0, 0)
      start_fetch(c1, 1)
      compute_chunk(c0, 0)
      wait_fetch(c1, 1)

      @pl.when(h < _N_CHUNKS // 2 - 1)
      def _():
        start_fetch(c1 + 1, 0)

      compute_chunk(c1, 1)
      return carry

    lax.fori_loop(0, _N_CHUNKS // 2, pair_body, 0)

    pltpu.sync_copy(pv_v, out_hbm.at[pl.ds(base, _P_PER_W)])

  return body(packed_table, probe_candidates, selection_logits)


_PB = 1024
_NB = _P_SPLIT // _PB


def _steer_tc(acc, pv, state, W1, b1, W2, b2, dirs, split, last):
  _DH = _D_MODEL // 2

  def body(acc_in_ref, state_ref, w1_ref, b1_ref, pv_ref, w2_ref, b2_ref,
           dirs_l_ref, dirs_r_ref, out_ref):
    i = pl.program_id(0)
    st = state_ref[...]
    z = jnp.sum(w1_ref[...] * st, axis=1) + b1_ref[0, :]
    h = 0.5 * z * (1.0 + lax.erf(z * jnp.float32(0.7071067811865476)))
    m = jnp.sum(w2_ref[...] * h[None, :], axis=1) + b2_ref[0, :]
    wvec = (pv_ref[0, :] * jax.nn.sigmoid(m))[None, :]

    @pl.when(i == 0)
    def _():
      out_ref[...] = acc_in_ref[...]

    out_ref[:, 0:_DH] += jnp.dot(wvec, dirs_l_ref[...],
                                 preferred_element_type=jnp.float32)
    out_ref[:, _DH:_D_MODEL] += jnp.dot(wvec, dirs_r_ref[...],
                                        preferred_element_type=jnp.float32)

    if last:
      @pl.when(i == _NB - 1)
      def _():
        out_ref[...] = jnp.tanh(out_ref[...])

  row0 = split * _NB

  return pl.pallas_call(
      body,
      grid=(_NB,),
      in_specs=[
          pl.BlockSpec((1, _D_MODEL), lambda i: (0, 0)),
          pl.BlockSpec((1, 4), lambda i: (0, 0)),
          pl.BlockSpec((32, 4), lambda i: (0, 0)),
          pl.BlockSpec((1, 32), lambda i: (0, 0)),
          pl.BlockSpec((1, _PB), lambda i: (0, i)),
          pl.BlockSpec((_PB, 32), lambda i: (row0 + i, 0)),
          pl.BlockSpec((1, _PB), lambda i: (0, row0 + i)),
          pl.BlockSpec((_PB, _DH), lambda i: (row0 + i, 0)),
          pl.BlockSpec((_PB, _DH), lambda i: (row0 + i, 1)),
      ],
      out_specs=pl.BlockSpec((1, _D_MODEL), lambda i: (0, 0)),
      out_shape=jax.ShapeDtypeStruct((1, _D_MODEL), jnp.float32),
      compiler_params=pltpu.CompilerParams(
          dimension_semantics=("arbitrary",)),
      cost_estimate=pl.CostEstimate(
          flops=2 * _P_SPLIT * _D_MODEL,
          bytes_accessed=4 * _P_SPLIT * _D_MODEL,
          transcendentals=0,
      ),
  )(acc, state.reshape(1, 4), W1, b1.reshape(1, 32), pv.reshape(1, _P_SPLIT),
    W2, b2.reshape(1, _N_PROBES), dirs, dirs)


def kernel(activations, state, probe_candidates, selection_logits,
           steering_dirs, W1, b1, W2, b2, scale):
  acts_bf = activations.astype(jnp.bfloat16)
  packed = lax.bitcast_convert_type(
      acts_bf.reshape(_N_FEAT // 2, 2), jnp.int32)
  acc = jnp.zeros((1, _D_MODEL), jnp.float32)
  for k in range(_N_SPLIT):
    pv = _probe_values_sc(packed, probe_candidates, selection_logits, k)
    acc = _steer_tc(acc, pv, state, W1, b1, W2, b2, steering_dirs,
                    split=k, last=(k == _N_SPLIT - 1))
  return acc.reshape(_D_MODEL) * (scale * 10.0)
